# baseline (speedup 1.0000x reference)
import functools

import jax
import jax.numpy as jnp
from jax import lax
from jax.experimental import pallas as pl
from jax.experimental.pallas import tpu as pltpu
from jax.experimental.pallas import tpu_sc as plsc

F32 = jnp.float32
BF16 = jnp.bfloat16
I32 = jnp.int32


def _dot16(a, b):
    return jnp.dot(a.astype(BF16), b.astype(BF16), preferred_element_type=F32)
H = 128
BN = 512
NSUB = 16
EBLK = 16384

NB, NP, NT = 100000, 50000, 1000
NPAD_B = 100352
NPAD_P = 50176
NPAD_T = 1024


def _ceil_to(x, m):
    return (x + m - 1) // m * m


def _zchunk(rows_per_tile, cap=1024):
    nzc = -(-rows_per_tile // cap)
    while rows_per_tile % nzc:
        nzc += 1
    return rows_per_tile // nzc, nzc


def _bn_affine(g, b, m, v):
    scale = g / jnp.sqrt(v + 1e-5)
    return jnp.stack([scale, b - m * scale])


def _enc_body(x_ref, wt_ref, b_ref, sc_ref, o_ref, *, f):
    x16 = x_ref[...].astype(BF16).astype(F32)
    w16 = wt_ref[...].astype(BF16).astype(F32)
    acc = x16[:, 0:1] * w16[0:1, :]
    for k in range(1, f):
        acc = acc + x16[:, k : k + 1] * w16[k : k + 1, :]
    acc = acc + b_ref[0:1, :]
    y = acc * sc_ref[0:1, :] + sc_ref[1:2, :]
    o_ref[...] = jnp.maximum(y, 0.0)


def _enc(x, W, b, bn4):
    n, f = x.shape
    return pl.pallas_call(
        functools.partial(_enc_body, f=f),
        grid=(n // BN,),
        in_specs=[
            pl.BlockSpec((BN, f), lambda i: (i, 0)),
            pl.BlockSpec((f, H), lambda i: (0, 0)),
            pl.BlockSpec((1, H), lambda i: (0, 0)),
            pl.BlockSpec((2, H), lambda i: (0, 0)),
        ],
        out_specs=pl.BlockSpec((BN, H), lambda i: (i, 0)),
        out_shape=jax.ShapeDtypeStruct((n, H), F32),
    )(x, W.T, b.reshape(1, H), _bn_affine(bn4[0], bn4[1], bn4[2], bn4[3]))


def _comb_std_body(*refs, nch, w):
    s = refs[0:nch]
    inv, xd, wlt, bl, wrt, scs, o = refs[nch:]
    iv = inv[:, 0:1]
    y = _dot16(xd[...], wrt[...])
    for fc in range(nch):
        y = y + _dot16(s[fc][...] * iv, wlt[fc * w : (fc + 1) * w, :])
    y = y + bl[0:1, :]
    y = y * scs[0:1, :] + scs[1:2, :]
    o[...] = jnp.maximum(y, 0.0)


def _comb_std(s8, inv, xd, Wl, bl, Wr, bnp):
    n = xd.shape[0]
    nch = len(s8)
    w = 128 // nch
    specs = [pl.BlockSpec((BN, w), lambda i: (i, 0)) for _ in range(nch)]
    specs += [
        pl.BlockSpec((BN, 16), lambda i: (i, 0)),
        pl.BlockSpec((BN, H), lambda i: (i, 0)),
        pl.BlockSpec((H, H), lambda i: (0, 0)),
        pl.BlockSpec((1, H), lambda i: (0, 0)),
        pl.BlockSpec((H, H), lambda i: (0, 0)),
        pl.BlockSpec((2, H), lambda i: (0, 0)),
    ]
    return pl.pallas_call(
        functools.partial(_comb_std_body, nch=nch, w=w),
        grid=(n // BN,),
        in_specs=specs,
        out_specs=pl.BlockSpec((BN, H), lambda i: (i, 0)),
        out_shape=jax.ShapeDtypeStruct((n, H), F32),
    )(*s8, inv, xd, Wl.T, bl.reshape(1, H), Wr.T,
      _bn_affine(bnp[0], bnp[1], bnp[2], bnp[3]))


def _comb_team_body(sA, sB, invA, invB, xd, wltA, blA, wrtA, wltB, blB, wrtB,
                    scs, o):
    y1 = _dot16(xd[...], wrtA[...])
    y2 = _dot16(xd[...], wrtB[...])
    y1 = y1 + _dot16(sA[...] * invA[:, 0:1], wltA[...])
    y2 = y2 + _dot16(sB[...] * invB[:, 0:1], wltB[...])
    y = 0.5 * ((y1 + blA[0:1, :]) + (y2 + blB[0:1, :]))
    y = y * scs[0:1, :] + scs[1:2, :]
    o[...] = jnp.maximum(y, 0.0)


def _comb_team(sA, cntA, sB, cntB, xd, WlA, blA, WrA, WlB, blB, WrB, bnp):
    n = xd.shape[0]
    hspec = pl.BlockSpec((BN, H), lambda i: (i, 0))
    cspec = pl.BlockSpec((BN, 16), lambda i: (i, 0))
    wspec = pl.BlockSpec((H, H), lambda i: (0, 0))
    bspec = pl.BlockSpec((1, H), lambda i: (0, 0))
    specs = [
        hspec, hspec, cspec, cspec, hspec,
        wspec, bspec, wspec,
        wspec, bspec, wspec,
        pl.BlockSpec((2, H), lambda i: (0, 0)),
    ]
    return pl.pallas_call(
        _comb_team_body,
        grid=(n // BN,),
        in_specs=specs,
        out_specs=pl.BlockSpec((BN, H), lambda i: (i, 0)),
        out_shape=jax.ShapeDtypeStruct((n, H), F32),
    )(sA, sB, cntA, cntB, xd, WlA.T, blA.reshape(1, H), WrA.T,
      WlB.T, blB.reshape(1, H), WrB.T,
      _bn_affine(bnp[0], bnp[1], bnp[2], bnp[3]))


def _mlp_body(x, w1t, b1, w2t, b2, w3, b3, o):
    h = jnp.maximum(_dot16(x[...], w1t[...]) + b1[0:1, :], 0.0)
    h = jnp.maximum(_dot16(h, w2t[...]) + b2[0:1, :], 0.0)
    h16 = h.astype(BF16).astype(F32)
    w316 = w3[0:1, :].astype(BF16).astype(F32)
    o[...] = jnp.sum(h16 * w316, axis=1, keepdims=True) + b3[0, 0]


def _mlp(x, pW1, pb1, pW2, pb2, pW3, pb3):
    n = x.shape[0]
    return pl.pallas_call(
        _mlp_body,
        grid=(n // BN,),
        in_specs=[
            pl.BlockSpec((BN, H), lambda i: (i, 0)),
            pl.BlockSpec((H, H), lambda i: (0, 0)),
            pl.BlockSpec((1, H), lambda i: (0, 0)),
            pl.BlockSpec((H, 64), lambda i: (0, 0)),
            pl.BlockSpec((1, 64), lambda i: (0, 0)),
            pl.BlockSpec((1, 64), lambda i: (0, 0)),
            pl.BlockSpec((1, 1), lambda i: (0, 0)),
        ],
        out_specs=pl.BlockSpec((BN, 1), lambda i: (i, 0)),
        out_shape=jax.ShapeDtypeStruct((n, 1), F32),
    )(x, pW1.T, pb1.reshape(1, H), pW2.T, pb2.reshape(1, 64),
      pW3.reshape(1, 64), pb3.reshape(1, 1))


def _pipe_pass(tab, src_r, dst_r, out, acc, sets, sems, *, s, ept, nblk, blke,
               r0, drpt, zc, nzc, zro):
    isem, gsem, ssem = sems
    ebase = s * ept

    def stage(b, si, di):
        e0 = ebase + b * blke
        pltpu.async_copy(src_r.at[pl.ds(e0, blke)], si, isem)
        pltpu.async_copy(dst_r.at[pl.ds(e0, blke)], di, isem)

    for i in range(nzc):
        pltpu.sync_copy(zro, acc.at[pl.ds(r0 + i * zc, zc)])
    plsc.subcore_barrier()
    stage(0, sets[0][0], sets[0][1])

    def blk2(b2, carry):
        for k in range(2):
            si, di, rw = sets[k]
            b = 2 * b2 + k
            pltpu.make_async_copy(src_r.at[pl.ds(0, blke)], si, isem).wait()
            pltpu.make_async_copy(dst_r.at[pl.ds(0, blke)], di, isem).wait()
            @pl.when(b2 >= 1)
            def _(rw=rw):
                pltpu.make_async_copy(tab.at[pl.ds(0, blke)], rw, ssem).wait()
            g = pltpu.async_copy(tab.at[si], rw, gsem)
            sn, dn, _rn = sets[1 - k]
            if k == 0:
                stage(b + 1, sn, dn)
            else:
                @pl.when(b2 < nblk // 2 - 1)
                def _(b=b, sn=sn, dn=dn):
                    stage(b + 1, sn, dn)
            g.wait()
            pltpu.async_copy(rw, acc.at[di], ssem, add=True)
        return carry

    lax.fori_loop(0, nblk // 2, blk2, 0)
    for k in range(2):
        pltpu.make_async_copy(tab.at[pl.ds(0, blke)], sets[k][2], ssem).wait()
    plsc.subcore_barrier()
    for i in range(nzc):
        pltpu.sync_copy(
            acc.at[pl.ds(r0 + i * zc, zc)], out.at[pl.ds(r0 + i * zc, zc)]
        )
    plsc.subcore_barrier()


def _segsum_call(n_pad, e_pad, w, blke):
    nch = 128 // w
    ept = e_pad // NSUB
    nblk = ept // blke
    drpt = n_pad // NSUB
    zc, nzc = _zchunk(drpt, cap=16384 // w)
    mesh = plsc.VectorSubcoreMesh(core_axis_name="c", subcore_axis_name="s")
    assert nblk % 2 == 0 and nblk >= 4 and nch % 2 == 0

    def body(*refs):
        tabs = refs[0:nch]
        src_r, dst_r, zro = refs[nch : nch + 3]
        outs = refs[nch + 3 : 2 * nch + 3]
        (sidx0, didx0, rows0, sidx1, didx1, rows1, acc,
         isem, gsem, ssem) = refs[2 * nch + 3 :]
        sets = ((sidx0, didx0, rows0), (sidx1, didx1, rows1))
        c = lax.axis_index("c")
        s = lax.axis_index("s")
        for fc in range(nch):
            @pl.when((fc % 2) == c)
            def _(fc=fc):
                _pipe_pass(tabs[fc], src_r, dst_r, outs[fc], acc, sets,
                           (isem, gsem, ssem), s=s, ept=ept, nblk=nblk,
                           blke=blke, r0=s * drpt, drpt=drpt, zc=zc, nzc=nzc,
                           zro=zro)

    return pl.kernel(
        body,
        out_type=[jax.ShapeDtypeStruct((n_pad, w), F32) for _ in range(nch)],
        mesh=mesh,
        scratch_types=[
            pltpu.VMEM((blke,), I32),
            pltpu.VMEM((blke,), I32),
            pltpu.VMEM((blke, w), F32),
            pltpu.VMEM((blke,), I32),
            pltpu.VMEM((blke,), I32),
            pltpu.VMEM((blke, w), F32),
            pltpu.VMEM_SHARED((n_pad, w), F32),
            pltpu.SemaphoreType.DMA,
            pltpu.SemaphoreType.DMA,
            pltpu.SemaphoreType.DMA,
        ],
        compiler_params=pltpu.CompilerParams(use_tc_tiling_on_sc=False),
    )


def _segsum(x, src1, dst1, n_pad, w, blke):
    nch = 128 // w
    tables = [x[:, k * w : (k + 1) * w] for k in range(nch)]
    zc, _ = _zchunk(n_pad // NSUB, cap=16384 // w)
    zeros = jnp.zeros((zc, w), F32)
    fn = _segsum_call(n_pad, src1.shape[0], w, blke)
    return fn(*tables, src1, dst1, zeros)


def _segsum_team_call(e_pad_a, e_pad_b):
    blke = 128
    drpt = NPAD_T // NSUB
    zc, nzc = _zchunk(drpt, cap=16384 // 128)
    mesh = plsc.VectorSubcoreMesh(core_axis_name="c", subcore_axis_name="s")

    def body(xb, xp, src_a, dst_a, src_b, dst_b, zro, out_a, out_b,
             sidx0, didx0, rows0, sidx1, didx1, rows1, acc, isem, gsem, ssem):
        sets = ((sidx0, didx0, rows0), (sidx1, didx1, rows1))
        c = lax.axis_index("c")
        s = lax.axis_index("s")
        common = dict(s=s, r0=s * drpt, drpt=drpt, zc=zc, nzc=nzc, zro=zro,
                      blke=blke)

        @pl.when(c == 0)
        def _():
            _pipe_pass(xb, src_a, dst_a, out_a, acc, sets, (isem, gsem, ssem),
                       ept=e_pad_a // NSUB, nblk=e_pad_a // NSUB // blke,
                       **common)

        @pl.when(c == 1)
        def _():
            _pipe_pass(xp, src_b, dst_b, out_b, acc, sets, (isem, gsem, ssem),
                       ept=e_pad_b // NSUB, nblk=e_pad_b // NSUB // blke,
                       **common)

    return pl.kernel(
        body,
        out_type=[
            jax.ShapeDtypeStruct((NPAD_T, 128), F32),
            jax.ShapeDtypeStruct((NPAD_T, 128), F32),
        ],
        mesh=mesh,
        scratch_types=[
            pltpu.VMEM((blke,), I32),
            pltpu.VMEM((blke,), I32),
            pltpu.VMEM((blke, 128), F32),
            pltpu.VMEM((blke,), I32),
            pltpu.VMEM((blke,), I32),
            pltpu.VMEM((blke, 128), F32),
            pltpu.VMEM_SHARED((NPAD_T, 128), F32),
            pltpu.SemaphoreType.DMA,
            pltpu.SemaphoreType.DMA,
            pltpu.SemaphoreType.DMA,
        ],
        compiler_params=pltpu.CompilerParams(use_tc_tiling_on_sc=False),
    )


def _segsum_team(xb, xp, srcA, dstA, srcB, dstB):
    zc, _ = _zchunk(NPAD_T // NSUB, cap=128)
    zeros = jnp.zeros((zc, 128), F32)
    fn = _segsum_team_call(srcA.shape[0], srcB.shape[0])
    return fn(xb, xp, srcA, dstA, srcB, dstB, zeros)


def _counts_call(n_pad_a, e_pad_a, n_pad_b, e_pad_b):
    mesh = plsc.VectorSubcoreMesh(core_axis_name="c", subcore_axis_name="s")
    acc_rows = max(n_pad_a, n_pad_b)
    zc_a, nzc_a = _zchunk(n_pad_a // NSUB)
    zc_b, nzc_b = _zchunk(n_pad_b // NSUB)
    zc_max = max(zc_a, zc_b)

    def body(dst_a, dst_b, ones_h, zro, out_a, out_b, didx, ones_v, acc, sem):
        c = lax.axis_index("c")
        s = lax.axis_index("s")
        pltpu.sync_copy(ones_h, ones_v)

        def one_type(dst_r, out, n_pad, e_pad, zc, nzc):
            ept = e_pad // NSUB
            nblk = ept // 256
            drpt = n_pad // NSUB
            r0 = s * drpt
            for i in range(nzc):
                pltpu.sync_copy(zro.at[pl.ds(0, zc)], acc.at[pl.ds(r0 + i * zc, zc)])
            plsc.subcore_barrier()

            def blk(b, carry):
                e0 = s * ept + b * 256
                pltpu.sync_copy(dst_r.at[pl.ds(e0, 256)], didx)
                pltpu.sync_copy(ones_v, acc.at[didx], add=True)
                return carry

            lax.fori_loop(0, nblk, blk, 0)
            plsc.subcore_barrier()
            for i in range(nzc):
                pltpu.sync_copy(
                    acc.at[pl.ds(r0 + i * zc, zc)], out.at[pl.ds(r0 + i * zc, zc)]
                )
            plsc.subcore_barrier()

        @pl.when(c == 0)
        def _():
            one_type(dst_a, out_a, n_pad_a, e_pad_a, zc_a, nzc_a)

        @pl.when(c == 1)
        def _():
            one_type(dst_b, out_b, n_pad_b, e_pad_b, zc_b, nzc_b)

    return pl.kernel(
        body,
        out_type=[
            jax.ShapeDtypeStruct((n_pad_a, 16), F32),
            jax.ShapeDtypeStruct((n_pad_b, 16), F32),
        ],
        mesh=mesh,
        scratch_types=[
            pltpu.VMEM((256,), I32),
            pltpu.VMEM((256, 16), F32),
            pltpu.VMEM_SHARED((acc_rows, 16), F32),
            pltpu.SemaphoreType.DMA,
        ],
        compiler_params=pltpu.CompilerParams(use_tc_tiling_on_sc=False),
    ), zc_max


def _counts(dst_a, n_pad_a, dst_b, n_pad_b):
    fn, zc_max = _counts_call(n_pad_a, dst_a.shape[0], n_pad_b, dst_b.shape[0])
    ones = jnp.ones((256, 16), F32)
    zeros = jnp.zeros((zc_max, 16), F32)
    return fn(dst_a, dst_b, ones, zeros)


def _pad_rows(x, n_pad):
    n, f = x.shape
    return jnp.concatenate([x, jnp.zeros((n_pad - n, f), x.dtype)], axis=0)


def _pad_edges(ei, n_dst, mult):
    e = ei.shape[1]
    ep = _ceil_to(e, mult)
    src = jnp.concatenate([ei[0].astype(I32), jnp.zeros((ep - e,), I32)])
    dst = jnp.concatenate([ei[1].astype(I32), jnp.full((ep - e,), n_dst, I32)])
    return src, dst


def kernel(x_batter, x_pitcher, x_team, inW_b, inb_b, inbn_b, inW_p, inb_p,
           inbn_p, inW_t, inb_t, inbn_t, conv_Wl, conv_bl, conv_Wr, bn_g,
           bn_b, bn_m, bn_v, pW1, pb1, pW2, pb2, pW3, pb3, ei_faces, ei_bb,
           ei_bt, ei_pt):
    srcF, dstF = _pad_edges(ei_faces, NP, 16384)
    srcB, dstB = _pad_edges(ei_bb, NB, 16384)
    srcBT, dstBT = _pad_edges(ei_bt, NT, 16384)
    srcPT, dstPT = _pad_edges(ei_pt, NT, 16384)

    xb = _enc(_pad_rows(x_batter, NPAD_B), inW_b, inb_b, inbn_b)
    xp = _enc(_pad_rows(x_pitcher, NPAD_P), inW_p, inb_p, inbn_p)
    xt = _enc(_pad_rows(x_team, NPAD_T), inW_t, inb_t, inbn_t)

    cntP, cntB = _counts(dstF, NPAD_P, dstB, NPAD_B)
    cntT1, cntT2 = _counts(dstBT, NPAD_T, dstPT, NPAD_T)
    cntP, cntB, cntT1, cntT2 = (
        1.0 / jnp.maximum(c, 1.0) for c in (cntP, cntB, cntT1, cntT2)
    )

    for i in range(3):
        sF4 = _segsum(xb, srcF, dstF, NPAD_P, 16, 512)
        sB8 = _segsum(xb, srcB, dstB, NPAD_B, 16, 512)
        sBT = jnp.concatenate(_segsum(xb, srcBT, dstBT, NPAD_T, 16, 512), axis=1)
        sPT = jnp.concatenate(_segsum(xp, srcPT, dstPT, NPAD_T, 16, 512), axis=1)

        bnp = lambda j: jnp.stack([bn_g[i, j], bn_b[i, j], bn_m[i, j], bn_v[i, j]])
        xp_new = _comb_std(sF4, cntP, xp, conv_Wl[i, 0], conv_bl[i, 0],
                           conv_Wr[i, 0], bnp(1))
        xb_new = _comb_std(sB8, cntB, xb, conv_Wl[i, 1], conv_bl[i, 1],
                           conv_Wr[i, 1], bnp(0))
        xt_new = _comb_team(sBT, cntT1, sPT, cntT2, xt,
                            conv_Wl[i, 2], conv_bl[i, 2], conv_Wr[i, 2],
                            conv_Wl[i, 3], conv_bl[i, 3], conv_Wr[i, 3],
                            bnp(2))
        xb, xp, xt = xb_new, xp_new, xt_new

    out = _mlp(xb, pW1, pb1, pW2, pb2, pW3, pb3)
    return out[:NB, 0]


# speedup vs baseline: 1.0001x; 1.0001x over previous
import functools

import jax
import jax.numpy as jnp
from jax import lax
from jax.experimental import pallas as pl
from jax.experimental.pallas import tpu as pltpu
from jax.experimental.pallas import tpu_sc as plsc

F32 = jnp.float32
BF16 = jnp.bfloat16
I32 = jnp.int32


def _dot16(a, b):
    return jnp.dot(a.astype(BF16), b.astype(BF16), preferred_element_type=F32)
H = 128
BN = 512
NSUB = 16
EBLK = 16384

NB, NP, NT = 100000, 50000, 1000
NPAD_B = 100352
NPAD_P = 50176
NPAD_T = 1024


def _ceil_to(x, m):
    return (x + m - 1) // m * m


def _zchunk(rows_per_tile, cap=1024):
    nzc = -(-rows_per_tile // cap)
    while rows_per_tile % nzc:
        nzc += 1
    return rows_per_tile // nzc, nzc


def _bn_affine(g, b, m, v):
    scale = g / jnp.sqrt(v + 1e-5)
    return jnp.stack([scale, b - m * scale])


def _enc_body(x_ref, wt_ref, b_ref, sc_ref, o_ref, *, f):
    x16 = x_ref[...].astype(BF16).astype(F32)
    w16 = wt_ref[...].astype(BF16).astype(F32)
    acc = x16[:, 0:1] * w16[0:1, :]
    for k in range(1, f):
        acc = acc + x16[:, k : k + 1] * w16[k : k + 1, :]
    acc = acc + b_ref[0:1, :]
    y = acc * sc_ref[0:1, :] + sc_ref[1:2, :]
    o_ref[...] = jnp.maximum(y, 0.0)


def _enc(x, W, b, bn4):
    n, f = x.shape
    return pl.pallas_call(
        functools.partial(_enc_body, f=f),
        grid=(n // BN,),
        in_specs=[
            pl.BlockSpec((BN, f), lambda i: (i, 0)),
            pl.BlockSpec((f, H), lambda i: (0, 0)),
            pl.BlockSpec((1, H), lambda i: (0, 0)),
            pl.BlockSpec((2, H), lambda i: (0, 0)),
        ],
        out_specs=pl.BlockSpec((BN, H), lambda i: (i, 0)),
        out_shape=jax.ShapeDtypeStruct((n, H), F32),
    )(x, W.T, b.reshape(1, H), _bn_affine(bn4[0], bn4[1], bn4[2], bn4[3]))


def _comb_std_body(*refs, nch, w):
    s = refs[0:nch]
    inv, xd, wlt, bl, wrt, scs, o = refs[nch:]
    iv = inv[:, 0:1]
    y = _dot16(xd[...], wrt[...])
    for fc in range(nch):
        y = y + _dot16(s[fc][...] * iv, wlt[fc * w : (fc + 1) * w, :])
    y = y + bl[0:1, :]
    y = y * scs[0:1, :] + scs[1:2, :]
    o[...] = jnp.maximum(y, 0.0)


def _comb_std(s8, inv, xd, Wl, bl, Wr, bnp):
    n = xd.shape[0]
    nch = len(s8)
    w = 128 // nch
    specs = [pl.BlockSpec((BN, w), lambda i: (i, 0)) for _ in range(nch)]
    specs += [
        pl.BlockSpec((BN, 16), lambda i: (i, 0)),
        pl.BlockSpec((BN, H), lambda i: (i, 0)),
        pl.BlockSpec((H, H), lambda i: (0, 0)),
        pl.BlockSpec((1, H), lambda i: (0, 0)),
        pl.BlockSpec((H, H), lambda i: (0, 0)),
        pl.BlockSpec((2, H), lambda i: (0, 0)),
    ]
    return pl.pallas_call(
        functools.partial(_comb_std_body, nch=nch, w=w),
        grid=(n // BN,),
        in_specs=specs,
        out_specs=pl.BlockSpec((BN, H), lambda i: (i, 0)),
        out_shape=jax.ShapeDtypeStruct((n, H), F32),
    )(*s8, inv, xd, Wl.T, bl.reshape(1, H), Wr.T,
      _bn_affine(bnp[0], bnp[1], bnp[2], bnp[3]))


def _comb_team_body(sA, sB, invA, invB, xd, wltA, blA, wrtA, wltB, blB, wrtB,
                    scs, o):
    y1 = _dot16(xd[...], wrtA[...])
    y2 = _dot16(xd[...], wrtB[...])
    y1 = y1 + _dot16(sA[...] * invA[:, 0:1], wltA[...])
    y2 = y2 + _dot16(sB[...] * invB[:, 0:1], wltB[...])
    y = 0.5 * ((y1 + blA[0:1, :]) + (y2 + blB[0:1, :]))
    y = y * scs[0:1, :] + scs[1:2, :]
    o[...] = jnp.maximum(y, 0.0)


def _comb_team(sA, cntA, sB, cntB, xd, WlA, blA, WrA, WlB, blB, WrB, bnp):
    n = xd.shape[0]
    hspec = pl.BlockSpec((BN, H), lambda i: (i, 0))
    cspec = pl.BlockSpec((BN, 16), lambda i: (i, 0))
    wspec = pl.BlockSpec((H, H), lambda i: (0, 0))
    bspec = pl.BlockSpec((1, H), lambda i: (0, 0))
    specs = [
        hspec, hspec, cspec, cspec, hspec,
        wspec, bspec, wspec,
        wspec, bspec, wspec,
        pl.BlockSpec((2, H), lambda i: (0, 0)),
    ]
    return pl.pallas_call(
        _comb_team_body,
        grid=(n // BN,),
        in_specs=specs,
        out_specs=pl.BlockSpec((BN, H), lambda i: (i, 0)),
        out_shape=jax.ShapeDtypeStruct((n, H), F32),
    )(sA, sB, cntA, cntB, xd, WlA.T, blA.reshape(1, H), WrA.T,
      WlB.T, blB.reshape(1, H), WrB.T,
      _bn_affine(bnp[0], bnp[1], bnp[2], bnp[3]))


def _mlp_body(x, w1t, b1, w2t, b2, w3, b3, o):
    h = jnp.maximum(_dot16(x[...], w1t[...]) + b1[0:1, :], 0.0)
    h = jnp.maximum(_dot16(h, w2t[...]) + b2[0:1, :], 0.0)
    h16 = h.astype(BF16).astype(F32)
    w316 = w3[0:1, :].astype(BF16).astype(F32)
    o[...] = jnp.sum(h16 * w316, axis=1, keepdims=True) + b3[0, 0]


def _mlp(x, pW1, pb1, pW2, pb2, pW3, pb3):
    n = x.shape[0]
    return pl.pallas_call(
        _mlp_body,
        grid=(n // BN,),
        in_specs=[
            pl.BlockSpec((BN, H), lambda i: (i, 0)),
            pl.BlockSpec((H, H), lambda i: (0, 0)),
            pl.BlockSpec((1, H), lambda i: (0, 0)),
            pl.BlockSpec((H, 64), lambda i: (0, 0)),
            pl.BlockSpec((1, 64), lambda i: (0, 0)),
            pl.BlockSpec((1, 64), lambda i: (0, 0)),
            pl.BlockSpec((1, 1), lambda i: (0, 0)),
        ],
        out_specs=pl.BlockSpec((BN, 1), lambda i: (i, 0)),
        out_shape=jax.ShapeDtypeStruct((n, 1), F32),
    )(x, pW1.T, pb1.reshape(1, H), pW2.T, pb2.reshape(1, 64),
      pW3.reshape(1, 64), pb3.reshape(1, 1))


def _pipe_pass(tab, src_r, dst_r, out, acc, sets, sems, *, s, ept, nblk, blke,
               r0, drpt, zc, nzc, zro):
    isem, gsem, ssem = sems
    ebase = s * ept

    def stage(b, si, di):
        e0 = ebase + b * blke
        pltpu.async_copy(src_r.at[pl.ds(e0, blke)], si, isem)
        pltpu.async_copy(dst_r.at[pl.ds(e0, blke)], di, isem)

    for i in range(nzc):
        pltpu.sync_copy(zro, acc.at[pl.ds(r0 + i * zc, zc)])
    plsc.subcore_barrier()
    stage(0, sets[0][0], sets[0][1])

    def blk2(b2, carry):
        for k in range(2):
            si, di, rw = sets[k]
            b = 2 * b2 + k
            pltpu.make_async_copy(src_r.at[pl.ds(0, blke)], si, isem).wait()
            pltpu.make_async_copy(dst_r.at[pl.ds(0, blke)], di, isem).wait()
            @pl.when(b2 >= 1)
            def _(rw=rw):
                pltpu.make_async_copy(tab.at[pl.ds(0, blke)], rw, ssem).wait()
            g = pltpu.async_copy(tab.at[si], rw, gsem)
            sn, dn, _rn = sets[1 - k]
            if k == 0:
                stage(b + 1, sn, dn)
            else:
                @pl.when(b2 < nblk // 2 - 1)
                def _(b=b, sn=sn, dn=dn):
                    stage(b + 1, sn, dn)
            g.wait()
            pltpu.async_copy(rw, acc.at[di], ssem, add=True)
        return carry

    lax.fori_loop(0, nblk // 2, blk2, 0)
    for k in range(2):
        pltpu.make_async_copy(tab.at[pl.ds(0, blke)], sets[k][2], ssem).wait()
    plsc.subcore_barrier()
    for i in range(nzc):
        pltpu.sync_copy(
            acc.at[pl.ds(r0 + i * zc, zc)], out.at[pl.ds(r0 + i * zc, zc)]
        )
    plsc.subcore_barrier()


def _segsum_call(n_pad, e_pad, w, blke):
    nch = 128 // w
    ept = e_pad // NSUB
    nblk = ept // blke
    drpt = n_pad // NSUB
    zc, nzc = _zchunk(drpt, cap=16384 // w)
    mesh = plsc.VectorSubcoreMesh(core_axis_name="c", subcore_axis_name="s")
    assert nblk % 2 == 0 and nblk >= 4 and nch % 2 == 0

    def body(*refs):
        tabs = refs[0:nch]
        src_r, dst_r, zro = refs[nch : nch + 3]
        outs = refs[nch + 3 : 2 * nch + 3]
        (sidx0, didx0, rows0, sidx1, didx1, rows1, acc,
         isem, gsem, ssem) = refs[2 * nch + 3 :]
        sets = ((sidx0, didx0, rows0), (sidx1, didx1, rows1))
        c = lax.axis_index("c")
        s = lax.axis_index("s")
        for fc in range(nch):
            @pl.when((fc % 2) == c)
            def _(fc=fc):
                _pipe_pass(tabs[fc], src_r, dst_r, outs[fc], acc, sets,
                           (isem, gsem, ssem), s=s, ept=ept, nblk=nblk,
                           blke=blke, r0=s * drpt, drpt=drpt, zc=zc, nzc=nzc,
                           zro=zro)

    return pl.kernel(
        body,
        out_type=[jax.ShapeDtypeStruct((n_pad, w), F32) for _ in range(nch)],
        mesh=mesh,
        scratch_types=[
            pltpu.VMEM((blke,), I32),
            pltpu.VMEM((blke,), I32),
            pltpu.VMEM((blke, w), F32),
            pltpu.VMEM((blke,), I32),
            pltpu.VMEM((blke,), I32),
            pltpu.VMEM((blke, w), F32),
            pltpu.VMEM_SHARED((n_pad, w), F32),
            pltpu.SemaphoreType.DMA,
            pltpu.SemaphoreType.DMA,
            pltpu.SemaphoreType.DMA,
        ],
        compiler_params=pltpu.CompilerParams(use_tc_tiling_on_sc=False),
    )


def _segsum(x, src1, dst1, n_pad, w, blke):
    nch = 128 // w
    tables = [x[:, k * w : (k + 1) * w] for k in range(nch)]
    zc, _ = _zchunk(n_pad // NSUB, cap=16384 // w)
    zeros = jnp.zeros((zc, w), F32)
    fn = _segsum_call(n_pad, src1.shape[0], w, blke)
    return fn(*tables, src1, dst1, zeros)


def _segsum_team_call(e_pad_a, e_pad_b):
    blke = 128
    drpt = NPAD_T // NSUB
    zc, nzc = _zchunk(drpt, cap=16384 // 128)
    mesh = plsc.VectorSubcoreMesh(core_axis_name="c", subcore_axis_name="s")

    def body(xb, xp, src_a, dst_a, src_b, dst_b, zro, out_a, out_b,
             sidx0, didx0, rows0, sidx1, didx1, rows1, acc, isem, gsem, ssem):
        sets = ((sidx0, didx0, rows0), (sidx1, didx1, rows1))
        c = lax.axis_index("c")
        s = lax.axis_index("s")
        common = dict(s=s, r0=s * drpt, drpt=drpt, zc=zc, nzc=nzc, zro=zro,
                      blke=blke)

        @pl.when(c == 0)
        def _():
            _pipe_pass(xb, src_a, dst_a, out_a, acc, sets, (isem, gsem, ssem),
                       ept=e_pad_a // NSUB, nblk=e_pad_a // NSUB // blke,
                       **common)

        @pl.when(c == 1)
        def _():
            _pipe_pass(xp, src_b, dst_b, out_b, acc, sets, (isem, gsem, ssem),
                       ept=e_pad_b // NSUB, nblk=e_pad_b // NSUB // blke,
                       **common)

    return pl.kernel(
        body,
        out_type=[
            jax.ShapeDtypeStruct((NPAD_T, 128), F32),
            jax.ShapeDtypeStruct((NPAD_T, 128), F32),
        ],
        mesh=mesh,
        scratch_types=[
            pltpu.VMEM((blke,), I32),
            pltpu.VMEM((blke,), I32),
            pltpu.VMEM((blke, 128), F32),
            pltpu.VMEM((blke,), I32),
            pltpu.VMEM((blke,), I32),
            pltpu.VMEM((blke, 128), F32),
            pltpu.VMEM_SHARED((NPAD_T, 128), F32),
            pltpu.SemaphoreType.DMA,
            pltpu.SemaphoreType.DMA,
            pltpu.SemaphoreType.DMA,
        ],
        compiler_params=pltpu.CompilerParams(use_tc_tiling_on_sc=False),
    )


def _segsum_team(xb, xp, srcA, dstA, srcB, dstB):
    zc, _ = _zchunk(NPAD_T // NSUB, cap=128)
    zeros = jnp.zeros((zc, 128), F32)
    fn = _segsum_team_call(srcA.shape[0], srcB.shape[0])
    return fn(xb, xp, srcA, dstA, srcB, dstB, zeros)


def _counts_call(n_pad_a, e_pad_a, n_pad_b, e_pad_b):
    mesh = plsc.VectorSubcoreMesh(core_axis_name="c", subcore_axis_name="s")
    acc_rows = max(n_pad_a, n_pad_b)
    zc_a, nzc_a = _zchunk(n_pad_a // NSUB)
    zc_b, nzc_b = _zchunk(n_pad_b // NSUB)
    zc_max = max(zc_a, zc_b)

    def body(dst_a, dst_b, ones_h, zro, out_a, out_b, didx, ones_v, acc, sem):
        c = lax.axis_index("c")
        s = lax.axis_index("s")
        pltpu.sync_copy(ones_h, ones_v)

        def one_type(dst_r, out, n_pad, e_pad, zc, nzc):
            ept = e_pad // NSUB
            nblk = ept // 256
            drpt = n_pad // NSUB
            r0 = s * drpt
            for i in range(nzc):
                pltpu.sync_copy(zro.at[pl.ds(0, zc)], acc.at[pl.ds(r0 + i * zc, zc)])
            plsc.subcore_barrier()

            def blk(b, carry):
                e0 = s * ept + b * 256
                pltpu.sync_copy(dst_r.at[pl.ds(e0, 256)], didx)
                pltpu.sync_copy(ones_v, acc.at[didx], add=True)
                return carry

            lax.fori_loop(0, nblk, blk, 0)
            plsc.subcore_barrier()
            for i in range(nzc):
                pltpu.sync_copy(
                    acc.at[pl.ds(r0 + i * zc, zc)], out.at[pl.ds(r0 + i * zc, zc)]
                )
            plsc.subcore_barrier()

        @pl.when(c == 0)
        def _():
            one_type(dst_a, out_a, n_pad_a, e_pad_a, zc_a, nzc_a)

        @pl.when(c == 1)
        def _():
            one_type(dst_b, out_b, n_pad_b, e_pad_b, zc_b, nzc_b)

    return pl.kernel(
        body,
        out_type=[
            jax.ShapeDtypeStruct((n_pad_a, 16), F32),
            jax.ShapeDtypeStruct((n_pad_b, 16), F32),
        ],
        mesh=mesh,
        scratch_types=[
            pltpu.VMEM((256,), I32),
            pltpu.VMEM((256, 16), F32),
            pltpu.VMEM_SHARED((acc_rows, 16), F32),
            pltpu.SemaphoreType.DMA,
        ],
        compiler_params=pltpu.CompilerParams(use_tc_tiling_on_sc=False),
    ), zc_max


def _counts(dst_a, n_pad_a, dst_b, n_pad_b):
    fn, zc_max = _counts_call(n_pad_a, dst_a.shape[0], n_pad_b, dst_b.shape[0])
    ones = jnp.ones((256, 16), F32)
    zeros = jnp.zeros((zc_max, 16), F32)
    return fn(dst_a, dst_b, ones, zeros)


def _pad_rows(x, n_pad):
    n, f = x.shape
    return jnp.concatenate([x, jnp.zeros((n_pad - n, f), x.dtype)], axis=0)


def _pad_edges(ei, n_dst, mult):
    e = ei.shape[1]
    ep = _ceil_to(e, mult)
    src = jnp.concatenate([ei[0].astype(I32), jnp.zeros((ep - e,), I32)])
    dst = jnp.concatenate([ei[1].astype(I32), jnp.full((ep - e,), n_dst, I32)])
    return src, dst


def kernel(x_batter, x_pitcher, x_team, inW_b, inb_b, inbn_b, inW_p, inb_p,
           inbn_p, inW_t, inb_t, inbn_t, conv_Wl, conv_bl, conv_Wr, bn_g,
           bn_b, bn_m, bn_v, pW1, pb1, pW2, pb2, pW3, pb3, ei_faces, ei_bb,
           ei_bt, ei_pt):
    srcF, dstF = _pad_edges(ei_faces, NP, 16384)
    srcB, dstB = _pad_edges(ei_bb, NB, 16384)
    srcBT, dstBT = _pad_edges(ei_bt, NT, 4096)
    srcPT, dstPT = _pad_edges(ei_pt, NT, 4096)

    xb = _enc(_pad_rows(x_batter, NPAD_B), inW_b, inb_b, inbn_b)
    xp = _enc(_pad_rows(x_pitcher, NPAD_P), inW_p, inb_p, inbn_p)
    xt = _enc(_pad_rows(x_team, NPAD_T), inW_t, inb_t, inbn_t)

    cntP, cntB = _counts(dstF, NPAD_P, dstB, NPAD_B)
    cntT1, cntT2 = _counts(dstBT, NPAD_T, dstPT, NPAD_T)
    cntP, cntB, cntT1, cntT2 = (
        1.0 / jnp.maximum(c, 1.0) for c in (cntP, cntB, cntT1, cntT2)
    )

    for i in range(3):
        sF4 = _segsum(xb, srcF, dstF, NPAD_P, 32, 256)
        sB8 = _segsum(xb, srcB, dstB, NPAD_B, 16, 512)
        sBT, sPT = _segsum_team(xb, xp, srcBT, dstBT, srcPT, dstPT)

        bnp = lambda j: jnp.stack([bn_g[i, j], bn_b[i, j], bn_m[i, j], bn_v[i, j]])
        xp_new = _comb_std(sF4, cntP, xp, conv_Wl[i, 0], conv_bl[i, 0],
                           conv_Wr[i, 0], bnp(1))
        xb_new = _comb_std(sB8, cntB, xb, conv_Wl[i, 1], conv_bl[i, 1],
                           conv_Wr[i, 1], bnp(0))
        xt_new = _comb_team(sBT, cntT1, sPT, cntT2, xt,
                            conv_Wl[i, 2], conv_bl[i, 2], conv_Wr[i, 2],
                            conv_Wl[i, 3], conv_bl[i, 3], conv_Wr[i, 3],
                            bnp(2))
        xb, xp, xt = xb_new, xp_new, xt_new

    out = _mlp(xb, pW1, pb1, pW2, pb2, pW3, pb3)
    return out[:NB, 0]


# speedup vs baseline: 1.1048x; 1.1046x over previous
import functools

import jax
import jax.numpy as jnp
from jax import lax
from jax.experimental import pallas as pl
from jax.experimental.pallas import tpu as pltpu
from jax.experimental.pallas import tpu_sc as plsc

F32 = jnp.float32
BF16 = jnp.bfloat16
I32 = jnp.int32


def _dot16(a, b):
    return jnp.dot(a.astype(BF16), b.astype(BF16), preferred_element_type=F32)
H = 128
BN = 512
NSUB = 16
EBLK = 16384

NB, NP, NT = 100000, 50000, 1000
NPAD_B = 100352
NPAD_P = 50176
NPAD_T = 1024


def _ceil_to(x, m):
    return (x + m - 1) // m * m


def _zchunk(rows_per_tile, cap=1024):
    nzc = -(-rows_per_tile // cap)
    while rows_per_tile % nzc:
        nzc += 1
    return rows_per_tile // nzc, nzc


def _bn_affine(g, b, m, v):
    scale = g / jnp.sqrt(v + 1e-5)
    return jnp.stack([scale, b - m * scale])


def _enc_body(x_ref, wt_ref, b_ref, sc_ref, o_ref, *, f):
    x16 = x_ref[...].astype(BF16).astype(F32)
    w16 = wt_ref[...].astype(BF16).astype(F32)
    acc = x16[:, 0:1] * w16[0:1, :]
    for k in range(1, f):
        acc = acc + x16[:, k : k + 1] * w16[k : k + 1, :]
    acc = acc + b_ref[0:1, :]
    y = acc * sc_ref[0:1, :] + sc_ref[1:2, :]
    o_ref[...] = jnp.maximum(y, 0.0)


def _enc(x, W, b, bn4):
    n, f = x.shape
    return pl.pallas_call(
        functools.partial(_enc_body, f=f),
        grid=(n // BN,),
        in_specs=[
            pl.BlockSpec((BN, f), lambda i: (i, 0)),
            pl.BlockSpec((f, H), lambda i: (0, 0)),
            pl.BlockSpec((1, H), lambda i: (0, 0)),
            pl.BlockSpec((2, H), lambda i: (0, 0)),
        ],
        out_specs=pl.BlockSpec((BN, H), lambda i: (i, 0)),
        out_shape=jax.ShapeDtypeStruct((n, H), F32),
    )(x, W.T, b.reshape(1, H), _bn_affine(bn4[0], bn4[1], bn4[2], bn4[3]))


def _comb_std_body(*refs, nch, w):
    s = refs[0:nch]
    inv, xd, wlt, bl, wrt, scs, o = refs[nch:]
    iv = inv[:, 0:1]
    y = _dot16(xd[...], wrt[...])
    for fc in range(nch):
        y = y + _dot16(s[fc][...] * iv, wlt[fc * w : (fc + 1) * w, :])
    y = y + bl[0:1, :]
    y = y * scs[0:1, :] + scs[1:2, :]
    o[...] = jnp.maximum(y, 0.0)


def _comb_std(s8, inv, xd, Wl, bl, Wr, bnp):
    n = xd.shape[0]
    nch = len(s8)
    w = 128 // nch
    specs = [pl.BlockSpec((BN, w), lambda i: (i, 0)) for _ in range(nch)]
    specs += [
        pl.BlockSpec((BN, 16), lambda i: (i, 0)),
        pl.BlockSpec((BN, H), lambda i: (i, 0)),
        pl.BlockSpec((H, H), lambda i: (0, 0)),
        pl.BlockSpec((1, H), lambda i: (0, 0)),
        pl.BlockSpec((H, H), lambda i: (0, 0)),
        pl.BlockSpec((2, H), lambda i: (0, 0)),
    ]
    return pl.pallas_call(
        functools.partial(_comb_std_body, nch=nch, w=w),
        grid=(n // BN,),
        in_specs=specs,
        out_specs=pl.BlockSpec((BN, H), lambda i: (i, 0)),
        out_shape=jax.ShapeDtypeStruct((n, H), F32),
    )(*s8, inv, xd, Wl.T, bl.reshape(1, H), Wr.T,
      _bn_affine(bnp[0], bnp[1], bnp[2], bnp[3]))


def _comb_team_body(sA, sB, invA, invB, xd, wltA, blA, wrtA, wltB, blB, wrtB,
                    scs, o):
    y1 = _dot16(xd[...], wrtA[...])
    y2 = _dot16(xd[...], wrtB[...])
    y1 = y1 + _dot16(sA[...] * invA[:, 0:1], wltA[...])
    y2 = y2 + _dot16(sB[...] * invB[:, 0:1], wltB[...])
    y = 0.5 * ((y1 + blA[0:1, :]) + (y2 + blB[0:1, :]))
    y = y * scs[0:1, :] + scs[1:2, :]
    o[...] = jnp.maximum(y, 0.0)


def _comb_team(sA, cntA, sB, cntB, xd, WlA, blA, WrA, WlB, blB, WrB, bnp):
    n = xd.shape[0]
    hspec = pl.BlockSpec((BN, H), lambda i: (i, 0))
    cspec = pl.BlockSpec((BN, 16), lambda i: (i, 0))
    wspec = pl.BlockSpec((H, H), lambda i: (0, 0))
    bspec = pl.BlockSpec((1, H), lambda i: (0, 0))
    specs = [
        hspec, hspec, cspec, cspec, hspec,
        wspec, bspec, wspec,
        wspec, bspec, wspec,
        pl.BlockSpec((2, H), lambda i: (0, 0)),
    ]
    return pl.pallas_call(
        _comb_team_body,
        grid=(n // BN,),
        in_specs=specs,
        out_specs=pl.BlockSpec((BN, H), lambda i: (i, 0)),
        out_shape=jax.ShapeDtypeStruct((n, H), F32),
    )(sA, sB, cntA, cntB, xd, WlA.T, blA.reshape(1, H), WrA.T,
      WlB.T, blB.reshape(1, H), WrB.T,
      _bn_affine(bnp[0], bnp[1], bnp[2], bnp[3]))


def _mlp_body(x, w1t, b1, w2t, b2, w3, b3, o):
    h = jnp.maximum(_dot16(x[...], w1t[...]) + b1[0:1, :], 0.0)
    h = jnp.maximum(_dot16(h, w2t[...]) + b2[0:1, :], 0.0)
    h16 = h.astype(BF16).astype(F32)
    w316 = w3[0:1, :].astype(BF16).astype(F32)
    o[...] = jnp.sum(h16 * w316, axis=1, keepdims=True) + b3[0, 0]


def _mlp(x, pW1, pb1, pW2, pb2, pW3, pb3):
    n = x.shape[0]
    return pl.pallas_call(
        _mlp_body,
        grid=(n // BN,),
        in_specs=[
            pl.BlockSpec((BN, H), lambda i: (i, 0)),
            pl.BlockSpec((H, H), lambda i: (0, 0)),
            pl.BlockSpec((1, H), lambda i: (0, 0)),
            pl.BlockSpec((H, 64), lambda i: (0, 0)),
            pl.BlockSpec((1, 64), lambda i: (0, 0)),
            pl.BlockSpec((1, 64), lambda i: (0, 0)),
            pl.BlockSpec((1, 1), lambda i: (0, 0)),
        ],
        out_specs=pl.BlockSpec((BN, 1), lambda i: (i, 0)),
        out_shape=jax.ShapeDtypeStruct((n, 1), F32),
    )(x, pW1.T, pb1.reshape(1, H), pW2.T, pb2.reshape(1, 64),
      pW3.reshape(1, 64), pb3.reshape(1, 1))


def _pipe_pass(tab, src_r, dst_r, out, acc, sets, sems, *, s, ept, nblk, blke,
               r0, drpt, zc, nzc, zro):
    isem, gsem, ssem = sems
    ebase = s * ept

    def stage(b, si, di):
        e0 = ebase + b * blke
        pltpu.async_copy(src_r.at[pl.ds(e0, blke)], si, isem)
        pltpu.async_copy(dst_r.at[pl.ds(e0, blke)], di, isem)

    for i in range(nzc):
        pltpu.sync_copy(zro, acc.at[pl.ds(r0 + i * zc, zc)])
    plsc.subcore_barrier()
    stage(0, sets[0][0], sets[0][1])

    def blk2(b2, carry):
        for k in range(2):
            si, di, rw = sets[k]
            b = 2 * b2 + k
            pltpu.make_async_copy(src_r.at[pl.ds(0, blke)], si, isem).wait()
            pltpu.make_async_copy(dst_r.at[pl.ds(0, blke)], di, isem).wait()
            @pl.when(b2 >= 1)
            def _(rw=rw):
                pltpu.make_async_copy(tab.at[pl.ds(0, blke)], rw, ssem).wait()
            g = pltpu.async_copy(tab.at[si], rw, gsem)
            sn, dn, _rn = sets[1 - k]
            if k == 0:
                stage(b + 1, sn, dn)
            else:
                @pl.when(b2 < nblk // 2 - 1)
                def _(b=b, sn=sn, dn=dn):
                    stage(b + 1, sn, dn)
            g.wait()
            pltpu.async_copy(rw, acc.at[di], ssem, add=True)
        return carry

    lax.fori_loop(0, nblk // 2, blk2, 0)
    for k in range(2):
        pltpu.make_async_copy(tab.at[pl.ds(0, blke)], sets[k][2], ssem).wait()
    plsc.subcore_barrier()
    for i in range(nzc):
        pltpu.sync_copy(
            acc.at[pl.ds(r0 + i * zc, zc)], out.at[pl.ds(r0 + i * zc, zc)]
        )
    plsc.subcore_barrier()


def _segsum_call(n_pad, e_pad, w, blke, nch, src_pat):
    ept = e_pad // NSUB
    nblk = ept // blke
    drpt = n_pad // NSUB
    zc, nzc = _zchunk(drpt, cap=16384 // w)
    mesh = plsc.VectorSubcoreMesh(core_axis_name="c", subcore_axis_name="s")
    assert nblk % 2 == 0 and nblk >= 4 and nch % 2 == 0

    def body(*refs):
        tabs = refs[0:nch]
        src0_r, src1_r, dst_r, zro = refs[nch : nch + 4]
        outs = refs[nch + 4 : 2 * nch + 4]
        (sidx0, didx0, rows0, sidx1, didx1, rows1, acc,
         isem, gsem, ssem) = refs[2 * nch + 4 :]
        sets = ((sidx0, didx0, rows0), (sidx1, didx1, rows1))
        srcs = (src0_r, src1_r)
        c = lax.axis_index("c")
        s = lax.axis_index("s")
        for fc in range(nch):
            @pl.when((fc % 2) == c)
            def _(fc=fc):
                _pipe_pass(tabs[fc], srcs[src_pat[fc]], dst_r, outs[fc], acc,
                           sets, (isem, gsem, ssem), s=s, ept=ept, nblk=nblk,
                           blke=blke, r0=s * drpt, drpt=drpt, zc=zc, nzc=nzc,
                           zro=zro)

    return pl.kernel(
        body,
        out_type=[jax.ShapeDtypeStruct((n_pad, w), F32) for _ in range(nch)],
        mesh=mesh,
        scratch_types=[
            pltpu.VMEM((blke,), I32),
            pltpu.VMEM((blke,), I32),
            pltpu.VMEM((blke, w), F32),
            pltpu.VMEM((blke,), I32),
            pltpu.VMEM((blke,), I32),
            pltpu.VMEM((blke, w), F32),
            pltpu.VMEM_SHARED((n_pad, w), F32),
            pltpu.SemaphoreType.DMA,
            pltpu.SemaphoreType.DMA,
            pltpu.SemaphoreType.DMA,
        ],
        compiler_params=pltpu.CompilerParams(use_tc_tiling_on_sc=False),
    )


def _segsum32(x32, src1, dst1, n_pad):
    zc, _ = _zchunk(n_pad // NSUB, cap=16384 // 32)
    zeros = jnp.zeros((zc, 32), F32)
    fn = _segsum_call(n_pad, src1.shape[0], 32, 256, 4, (0, 0, 0, 0))
    return fn(*x32, src1, src1, dst1, zeros)


def _segsum16v(x32, src2, src2p1, dst1, n_pad):
    views = [t.reshape(-1, 16) for t in x32]
    tabs = [views[fc // 2] for fc in range(8)]
    zc, _ = _zchunk(n_pad // NSUB, cap=16384 // 16)
    zeros = jnp.zeros((zc, 16), F32)
    fn = _segsum_call(n_pad, dst1.shape[0], 16, 512, 8,
                      (0, 1, 0, 1, 0, 1, 0, 1))
    return fn(*tabs, src2, src2p1, dst1, zeros)


def _segsum_team_call(e_pad_a, e_pad_b):
    blke = 128
    drpt = NPAD_T // NSUB
    zc, nzc = _zchunk(drpt, cap=16384 // 128)
    mesh = plsc.VectorSubcoreMesh(core_axis_name="c", subcore_axis_name="s")

    def body(xb, xp, src_a, dst_a, src_b, dst_b, zro, out_a, out_b,
             sidx0, didx0, rows0, sidx1, didx1, rows1, acc, isem, gsem, ssem):
        sets = ((sidx0, didx0, rows0), (sidx1, didx1, rows1))
        c = lax.axis_index("c")
        s = lax.axis_index("s")
        common = dict(s=s, r0=s * drpt, drpt=drpt, zc=zc, nzc=nzc, zro=zro,
                      blke=blke)

        @pl.when(c == 0)
        def _():
            _pipe_pass(xb, src_a, dst_a, out_a, acc, sets, (isem, gsem, ssem),
                       ept=e_pad_a // NSUB, nblk=e_pad_a // NSUB // blke,
                       **common)

        @pl.when(c == 1)
        def _():
            _pipe_pass(xp, src_b, dst_b, out_b, acc, sets, (isem, gsem, ssem),
                       ept=e_pad_b // NSUB, nblk=e_pad_b // NSUB // blke,
                       **common)

    return pl.kernel(
        body,
        out_type=[
            jax.ShapeDtypeStruct((NPAD_T, 128), F32),
            jax.ShapeDtypeStruct((NPAD_T, 128), F32),
        ],
        mesh=mesh,
        scratch_types=[
            pltpu.VMEM((blke,), I32),
            pltpu.VMEM((blke,), I32),
            pltpu.VMEM((blke, 128), F32),
            pltpu.VMEM((blke,), I32),
            pltpu.VMEM((blke,), I32),
            pltpu.VMEM((blke, 128), F32),
            pltpu.VMEM_SHARED((NPAD_T, 128), F32),
            pltpu.SemaphoreType.DMA,
            pltpu.SemaphoreType.DMA,
            pltpu.SemaphoreType.DMA,
        ],
        compiler_params=pltpu.CompilerParams(use_tc_tiling_on_sc=False),
    )


def _segsum_team(xb, xp, srcA, dstA, srcB, dstB):
    zc, _ = _zchunk(NPAD_T // NSUB, cap=128)
    zeros = jnp.zeros((zc, 128), F32)
    fn = _segsum_team_call(srcA.shape[0], srcB.shape[0])
    return fn(xb, xp, srcA, dstA, srcB, dstB, zeros)


def _counts_call(n_pad_a, e_pad_a, n_pad_b, e_pad_b):
    mesh = plsc.VectorSubcoreMesh(core_axis_name="c", subcore_axis_name="s")
    acc_rows = max(n_pad_a, n_pad_b)
    zc_a, nzc_a = _zchunk(n_pad_a // NSUB)
    zc_b, nzc_b = _zchunk(n_pad_b // NSUB)
    zc_max = max(zc_a, zc_b)

    def body(dst_a, dst_b, ones_h, zro, out_a, out_b, didx, ones_v, acc, sem):
        c = lax.axis_index("c")
        s = lax.axis_index("s")
        pltpu.sync_copy(ones_h, ones_v)

        def one_type(dst_r, out, n_pad, e_pad, zc, nzc):
            ept = e_pad // NSUB
            nblk = ept // 256
            drpt = n_pad // NSUB
            r0 = s * drpt
            for i in range(nzc):
                pltpu.sync_copy(zro.at[pl.ds(0, zc)], acc.at[pl.ds(r0 + i * zc, zc)])
            plsc.subcore_barrier()

            def blk(b, carry):
                e0 = s * ept + b * 256
                pltpu.sync_copy(dst_r.at[pl.ds(e0, 256)], didx)
                pltpu.sync_copy(ones_v, acc.at[didx], add=True)
                return carry

            lax.fori_loop(0, nblk, blk, 0)
            plsc.subcore_barrier()
            for i in range(nzc):
                pltpu.sync_copy(
                    acc.at[pl.ds(r0 + i * zc, zc)], out.at[pl.ds(r0 + i * zc, zc)]
                )
            plsc.subcore_barrier()

        @pl.when(c == 0)
        def _():
            one_type(dst_a, out_a, n_pad_a, e_pad_a, zc_a, nzc_a)

        @pl.when(c == 1)
        def _():
            one_type(dst_b, out_b, n_pad_b, e_pad_b, zc_b, nzc_b)

    return pl.kernel(
        body,
        out_type=[
            jax.ShapeDtypeStruct((n_pad_a, 16), F32),
            jax.ShapeDtypeStruct((n_pad_b, 16), F32),
        ],
        mesh=mesh,
        scratch_types=[
            pltpu.VMEM((256,), I32),
            pltpu.VMEM((256, 16), F32),
            pltpu.VMEM_SHARED((acc_rows, 16), F32),
            pltpu.SemaphoreType.DMA,
        ],
        compiler_params=pltpu.CompilerParams(use_tc_tiling_on_sc=False),
    ), zc_max


def _counts(dst_a, n_pad_a, dst_b, n_pad_b):
    fn, zc_max = _counts_call(n_pad_a, dst_a.shape[0], n_pad_b, dst_b.shape[0])
    ones = jnp.ones((256, 16), F32)
    zeros = jnp.zeros((zc_max, 16), F32)
    return fn(dst_a, dst_b, ones, zeros)


def _pad_rows(x, n_pad):
    n, f = x.shape
    return jnp.concatenate([x, jnp.zeros((n_pad - n, f), x.dtype)], axis=0)


def _pad_edges(ei, n_dst, mult):
    e = ei.shape[1]
    ep = _ceil_to(e, mult)
    src = jnp.concatenate([ei[0].astype(I32), jnp.zeros((ep - e,), I32)])
    dst = jnp.concatenate([ei[1].astype(I32), jnp.full((ep - e,), n_dst, I32)])
    return src, dst


def kernel(x_batter, x_pitcher, x_team, inW_b, inb_b, inbn_b, inW_p, inb_p,
           inbn_p, inW_t, inb_t, inbn_t, conv_Wl, conv_bl, conv_Wr, bn_g,
           bn_b, bn_m, bn_v, pW1, pb1, pW2, pb2, pW3, pb3, ei_faces, ei_bb,
           ei_bt, ei_pt):
    srcF, dstF = _pad_edges(ei_faces, NP, 16384)
    srcB, dstB = _pad_edges(ei_bb, NB, 16384)
    srcBT, dstBT = _pad_edges(ei_bt, NT, 4096)
    srcPT, dstPT = _pad_edges(ei_pt, NT, 4096)

    xb = _enc(_pad_rows(x_batter, NPAD_B), inW_b, inb_b, inbn_b)
    xp = _enc(_pad_rows(x_pitcher, NPAD_P), inW_p, inb_p, inbn_p)
    xt = _enc(_pad_rows(x_team, NPAD_T), inW_t, inb_t, inbn_t)

    cntP, cntB = _counts(dstF, NPAD_P, dstB, NPAD_B)
    cntT1, cntT2 = _counts(dstBT, NPAD_T, dstPT, NPAD_T)
    cntP, cntB, cntT1, cntT2 = (
        1.0 / jnp.maximum(c, 1.0) for c in (cntP, cntB, cntT1, cntT2)
    )

    srcB2 = srcB * 2
    srcB2p1 = srcB2 + 1

    for i in range(3):
        x32 = [xb[:, k * 32 : (k + 1) * 32] for k in range(4)]
        sF4 = _segsum32(x32, srcF, dstF, NPAD_P)
        sB8 = _segsum16v(x32, srcB2, srcB2p1, dstB, NPAD_B)
        sBT, sPT = _segsum_team(xb, xp, srcBT, dstBT, srcPT, dstPT)

        bnp = lambda j: jnp.stack([bn_g[i, j], bn_b[i, j], bn_m[i, j], bn_v[i, j]])
        xp_new = _comb_std(sF4, cntP, xp, conv_Wl[i, 0], conv_bl[i, 0],
                           conv_Wr[i, 0], bnp(1))
        xb_new = _comb_std(sB8, cntB, xb, conv_Wl[i, 1], conv_bl[i, 1],
                           conv_Wr[i, 1], bnp(0))
        xt_new = _comb_team(sBT, cntT1, sPT, cntT2, xt,
                            conv_Wl[i, 2], conv_bl[i, 2], conv_Wr[i, 2],
                            conv_Wl[i, 3], conv_bl[i, 3], conv_Wr[i, 3],
                            bnp(2))
        xb, xp, xt = xb_new, xp_new, xt_new

    out = _mlp(xb, pW1, pb1, pW2, pb2, pW3, pb3)
    return out[:NB, 0]
